# baseline (speedup 1.0000x reference)
import functools

import jax
import jax.numpy as jnp
import numpy as np
from jax import lax
from jax.experimental import pallas as pl
from jax.experimental.pallas import tpu as pltpu
from jax.experimental.pallas import tpu_sc as plsc

N = 10000
E = 160000
D_IN = 256
D = 512
HEADS = 8
DH = 64
LAYERS = 2

NC = 2
NS = 16
NW = NC * NS
ACC = 10240
RPT = ACC // NW
MAXNB = 96
EPAD = MAXNB * 128

_MESH = dict(core_axis_name="c", subcore_axis_name="s", num_cores=NC,
             num_subcores=NS)


def _fill_vmem(ref, rows, cols, value):
    nseg = cols // 16

    def body(i, _):
        r = i // nseg
        cc = (i % nseg) * 16
        ref[r, pl.ds(cc, 16)] = jnp.full((16,), value, jnp.float32)
        return 0

    lax.fori_loop(0, rows * nseg, body, 0)


def _make_sorted_sum(C, mode):
    scratch = [
        pltpu.VMEM((16,), jnp.int32),
        pltpu.VMEM((EPAD,), jnp.int32),
        pltpu.VMEM((RPT + 16, C), jnp.float32),
    ]
    if mode == "gather":
        scratch.append(pltpu.VMEM((EPAD,), jnp.int32))
    if mode != "ones":
        scratch += [pltpu.VMEM((128, C), jnp.float32),
                    pltpu.VMEM((128, C), jnp.float32),
                    pltpu.SemaphoreType.DMA,
                    pltpu.SemaphoreType.DMA]

    def body(*refs):
        if mode == "gather":
            (tbl_hbm, srcs_hbm, dsts_hbm, params_hbm, out_hbm,
             pbuf, idxd, acc, idxs, buf0, buf1, sem0, sem1) = refs
        elif mode == "linear":
            (vals_hbm, dsts_hbm, params_hbm, out_hbm,
             pbuf, idxd, acc, buf0, buf1, sem0, sem1) = refs
        else:
            (dsts_hbm, params_hbm, out_hbm, pbuf, idxd, acc) = refs
        c = lax.axis_index("c")
        s = lax.axis_index("s")
        w = c * NS + s
        base = w * RPT
        pltpu.sync_copy(params_hbm.at[w], pbuf)
        pv = pbuf[...]
        aw = pl.multiple_of(pv[0], 128)
        nb = pv[1]
        _fill_vmem(acc, RPT + 16, C, 0.0)
        pltpu.sync_copy(dsts_hbm.at[pl.ds(aw, EPAD)], idxd)
        if mode == "gather":
            pltpu.sync_copy(srcs_hbm.at[pl.ds(aw, EPAD)], idxs)

        ones = jnp.full((16,), 1.0, jnp.float32)

        def process(b, rbuf):
            for g in range(8):
                dstv = idxd[pl.ds(pl.multiple_of(b * 128 + g * 16, 16), 16)]
                rel = dstv - base
                ok = (rel >= 0) & (rel < RPT)
                rr = jnp.where(ok, rel, RPT)
                if mode == "ones":
                    for j in range(16):
                        plsc.addupdate(acc.at[rr[j], pl.ds(0, 16)], ones)
                else:
                    for j in range(0, 16, 2):
                        r0, r1 = rr[j], rr[j + 1]
                        vals0 = [rbuf[g * 16 + j, pl.ds(seg * 16, 16)]
                                 for seg in range(C // 16)]
                        vals1 = [rbuf[g * 16 + j + 1, pl.ds(seg * 16, 16)]
                                 for seg in range(C // 16)]
                        for seg in range(C // 16):
                            plsc.addupdate(acc.at[r0, pl.ds(seg * 16, 16)],
                                           vals0[seg])
                        for seg in range(C // 16):
                            plsc.addupdate(acc.at[r1, pl.ds(seg * 16, 16)],
                                           vals1[seg])

        if mode == "ones":
            def bbody(b, _):
                process(b, None)
                return 0
            lax.fori_loop(0, nb, bbody, 0)
        else:
            bufs = (buf0, buf1)
            sems = (sem0, sem1)

            def mk(b, p):
                if mode == "gather":
                    return pltpu.make_async_copy(
                        tbl_hbm.at[idxs.at[pl.ds(pl.multiple_of(b * 128, 128),
                                                 128)]],
                        bufs[p], sems[p])
                return pltpu.make_async_copy(
                    vals_hbm.at[pl.ds(pl.multiple_of(aw + b * 128, 128),
                                      128), :],
                    bufs[p], sems[p])

            @pl.when(nb > 0)
            def _():
                mk(0, 0).start()

            @pl.when(nb > 1)
            def _():
                mk(1, 1).start()

            def bbody(i, _):
                b0 = 2 * i
                b1 = b0 + 1

                @pl.when(b0 < nb)
                def _():
                    mk(b0, 0).wait()
                    process(b0, bufs[0])

                    @pl.when(b0 + 2 < nb)
                    def _():
                        mk(b0 + 2, 0).start()

                @pl.when(b1 < nb)
                def _():
                    mk(b1, 1).wait()
                    process(b1, bufs[1])

                    @pl.when(b1 + 2 < nb)
                    def _():
                        mk(b1 + 2, 1).start()

                return 0

            lax.fori_loop(0, (MAXNB + 1) // 2, bbody, 0)

        pltpu.sync_copy(acc.at[pl.ds(0, RPT), :],
                        out_hbm.at[pl.ds(w * RPT, RPT), :])

    def outs(_):
        return jax.ShapeDtypeStruct((ACC, C), jnp.float32)

    return pl.kernel(
        body,
        out_type=jax.ShapeDtypeStruct((ACC, C), jnp.float32),
        mesh=plsc.VectorSubcoreMesh(**_MESH),
        scratch_types=scratch,
    )


_make_sorted_sum = functools.lru_cache(maxsize=None)(_make_sorted_sum)


def _sc_deg(dst_s, params):
    return _make_sorted_sum(16, "ones")(dst_s, params)


def _sc_gather_sum(tbl, src_s, dst_s, params):
    return _make_sorted_sum(128, "gather")(tbl, src_s, dst_s, params)


def _sc_scatter_sum(vals, dst_s, params):
    return _make_sorted_sum(vals.shape[1], "linear")(vals, dst_s, params)


_GB = 40
_GNB = 125
_EPW = 5000


@functools.lru_cache(maxsize=None)
def _gather3_kernel():
    def body(q_hbm, k_hbm, v_hbm, srcs_hbm, dsts_hbm,
             qd_hbm, ks_hbm, vs_hbm,
             idxs, idxd, b0, b1, b2, b3,
             g0, g1, g2, g3, w0, w1, w2, w3):
        c = lax.axis_index("c")
        s = lax.axis_index("s")
        w = c * NS + s
        e0 = pl.multiple_of(w * _EPW, 8)
        pltpu.sync_copy(srcs_hbm.at[pl.ds(e0, _EPW)], idxs)
        pltpu.sync_copy(dsts_hbm.at[pl.ds(e0, _EPW)], idxd)
        bufs = (b0, b1, b2, b3)
        gsems = (g0, g1, g2, g3)
        wsems = (w0, w1, w2, w3)

        def one_pass(tbl, idx, out):
            def mkg(b, p):
                return pltpu.make_async_copy(
                    tbl.at[idx.at[pl.ds(pl.multiple_of(b * _GB, 8), _GB)]],
                    bufs[p], gsems[p])

            def mkw(b, p):
                return pltpu.make_async_copy(
                    bufs[p],
                    out.at[pl.ds(pl.multiple_of(e0 + b * _GB, 8), _GB), :],
                    wsems[p])

            mkg(0, 0).start()
            mkg(1, 1).start()

            def bbody(i, _):
                for k4 in range(4):
                    b = 4 * i + k4
                    p = k4

                    @pl.when(b < _GNB)
                    def _():
                        mkg(b, p).wait()
                        mkw(b, p).start()
                        nxt = b + 2
                        pn = (k4 + 2) % 4

                        @pl.when(nxt < _GNB)
                        def _():

                            @pl.when(nxt >= 4)
                            def _():
                                mkw(nxt - 4, pn).wait()

                            mkg(nxt, pn).start()

                return 0

            lax.fori_loop(0, (_GNB + 3) // 4, bbody, 0)
            for tail in range(4):
                b = _GNB - 4 + tail
                mkw(b, b % 4).wait()

        one_pass(q_hbm, idxd, qd_hbm)
        one_pass(k_hbm, idxs, ks_hbm)
        one_pass(v_hbm, idxs, vs_hbm)

    return pl.kernel(
        body,
        out_type=(jax.ShapeDtypeStruct((E, D // 2), jnp.int32),
                  jax.ShapeDtypeStruct((E, D // 2), jnp.int32),
                  jax.ShapeDtypeStruct((E, D // 2), jnp.int32)),
        mesh=plsc.VectorSubcoreMesh(**_MESH),
        scratch_types=[
            pltpu.VMEM((_EPW,), jnp.int32),
            pltpu.VMEM((_EPW,), jnp.int32),
        ] + [pltpu.VMEM((_GB, D // 2), jnp.int32)] * 4
        + [pltpu.SemaphoreType.DMA] * 8,
    )


def _sc_gather3(q, k, v, src_s, dst_s):
    return _gather3_kernel()(q, k, v, src_s, dst_s)


BLK = 2000
BLK7 = 1000
EBLK = 2000


def _ln(x, g, b):
    mu = jnp.mean(x, axis=-1, keepdims=True)
    xc = x - mu
    var = jnp.mean(xc * xc, axis=-1, keepdims=True)
    return xc * lax.rsqrt(var + 1e-5) * g + b


def _dot(a, b):
    return jnp.dot(a, b, preferred_element_type=jnp.float32)


def _t1_body(x_ref, w_ref, degp_ref, c0, c1, c2, c3, dinv_ref):
    xw = _dot(x_ref[...], w_ref[...])
    deg = degp_ref[:, 0:1] + 1.0
    dinv = lax.rsqrt(deg)
    hwp = xw * dinv
    outs = (c0, c1, c2, c3)
    for j in range(4):
        outs[j][...] = hwp[:, j * 128:(j + 1) * 128]
    dinv_ref[...] = dinv


def _t1(x, W1, degp):
    return pl.pallas_call(
        _t1_body,
        grid=(N // BLK,),
        in_specs=[
            pl.BlockSpec((BLK, D_IN), lambda i: (i, 0)),
            pl.BlockSpec((D_IN, D), lambda i: (0, 0)),
            pl.BlockSpec((BLK, 16), lambda i: (i, 0)),
        ],
        out_specs=[pl.BlockSpec((BLK, 128), lambda i: (i, 0))] * 4
        + [pl.BlockSpec((BLK, 1), lambda i: (i, 0))],
        out_shape=[jax.ShapeDtypeStruct((N, 128), jnp.float32)] * 4
        + [jax.ShapeDtypeStruct((N, 1), jnp.float32)],
    )(x, W1, degp)


def _t3_body(h_ref, w_ref, dinv_ref, c0, c1, c2, c3):
    hw = _dot(h_ref[...], w_ref[...])
    hwp = hw * dinv_ref[...]
    outs = (c0, c1, c2, c3)
    for j in range(4):
        outs[j][...] = hwp[:, j * 128:(j + 1) * 128]


def _t3(h, W2, dinv):
    return pl.pallas_call(
        _t3_body,
        grid=(N // BLK,),
        in_specs=[
            pl.BlockSpec((BLK, D), lambda i: (i, 0)),
            pl.BlockSpec((D, D), lambda i: (0, 0)),
            pl.BlockSpec((BLK, 1), lambda i: (i, 0)),
        ],
        out_specs=[pl.BlockSpec((BLK, 128), lambda i: (i, 0))] * 4,
        out_shape=[jax.ShapeDtypeStruct((N, 128), jnp.float32)] * 4,
    )(h, W2, dinv)


def _t2_body(a0, a1, a2, a3, c0, c1, c2, c3, dinv_ref, b_ref, out_ref):
    agg = jnp.concatenate([r[...] for r in (a0, a1, a2, a3)], axis=1)
    hwp = jnp.concatenate([r[...] for r in (c0, c1, c2, c3)], axis=1)
    out_ref[...] = jax.nn.gelu(dinv_ref[...] * (agg + hwp) + b_ref[...])


def _t2(agg_chunks, hwp_chunks, dinv, bias):
    return pl.pallas_call(
        _t2_body,
        grid=(N // BLK,),
        in_specs=[pl.BlockSpec((BLK, 128), lambda i: (i, 0))] * 8
        + [
            pl.BlockSpec((BLK, 1), lambda i: (i, 0)),
            pl.BlockSpec((1, D), lambda i: (0, 0)),
        ],
        out_specs=pl.BlockSpec((BLK, D), lambda i: (i, 0)),
        out_shape=jax.ShapeDtypeStruct((N, D), jnp.float32),
    )(*agg_chunks, *hwp_chunks, dinv, bias)


def _t5_body(h_ref, wq_ref, wk_ref, wv_ref, bd_ref, q_ref, k_ref, v_ref,
             es_ref, qb_ref, kb_ref, vb_ref):
    h = h_ref[...]
    q = _dot(h, wq_ref[...])
    k = _dot(h, wk_ref[...])
    v = _dot(h, wv_ref[...])
    q_ref[...] = q
    k_ref[...] = k
    v_ref[...] = v
    es_ref[...] = jnp.exp(_dot(q * k, bd_ref[...]) * 0.125)
    qb_ref[...] = q.astype(jnp.bfloat16)
    kb_ref[...] = k.astype(jnp.bfloat16)
    vb_ref[...] = v.astype(jnp.bfloat16)


def _t5(h, Wq, Wk, Wv, bd):
    return pl.pallas_call(
        _t5_body,
        grid=(N // BLK,),
        in_specs=[
            pl.BlockSpec((BLK, D), lambda i: (i, 0)),
            pl.BlockSpec((D, D), lambda i: (0, 0)),
            pl.BlockSpec((D, D), lambda i: (0, 0)),
            pl.BlockSpec((D, D), lambda i: (0, 0)),
            pl.BlockSpec((D, 8), lambda i: (0, 0)),
        ],
        out_specs=[pl.BlockSpec((BLK, D), lambda i: (i, 0))] * 3
        + [pl.BlockSpec((BLK, 8), lambda i: (i, 0))]
        + [pl.BlockSpec((BLK, D), lambda i: (i, 0))] * 3,
        out_shape=[jax.ShapeDtypeStruct((N, D), jnp.float32)] * 3
        + [jax.ShapeDtypeStruct((N, 8), jnp.float32)]
        + [jax.ShapeDtypeStruct((N, D), jnp.bfloat16)] * 3,
    )(h, Wq, Wk, Wv, bd)


def _t6_body(qd_ref, ks_ref, vs_ref, bd_ref, ex_ref, w0, w1, w2, w3):
    prod = (qd_ref[...].astype(jnp.float32)
            * ks_ref[...].astype(jnp.float32))
    ex = jnp.exp(_dot(prod, bd_ref[...]) * 0.125)
    ex_ref[...] = jnp.concatenate([ex, ex], axis=1)
    vs = vs_ref[...].astype(jnp.float32)
    outs = (w0, w1, w2, w3)
    for j in range(4):
        b = j * 128
        outs[j][...] = jnp.concatenate(
            [vs[:, b:b + 64] * ex[:, 2 * j:2 * j + 1],
             vs[:, b + 64:b + 128] * ex[:, 2 * j + 1:2 * j + 2]], axis=1)


def _t6(qd, ks, vs, bd):
    return pl.pallas_call(
        _t6_body,
        grid=(E // EBLK,),
        in_specs=[
            pl.BlockSpec((EBLK, D), lambda i: (i, 0)),
            pl.BlockSpec((EBLK, D), lambda i: (i, 0)),
            pl.BlockSpec((EBLK, D), lambda i: (i, 0)),
            pl.BlockSpec((D, 8), lambda i: (0, 0)),
        ],
        out_specs=[pl.BlockSpec((EBLK, 16), lambda i: (i, 0))]
        + [pl.BlockSpec((EBLK, 128), lambda i: (i, 0))] * 4,
        out_shape=[jax.ShapeDtypeStruct((E, 16), jnp.float32)]
        + [jax.ShapeDtypeStruct((E, 128), jnp.float32)] * 4,
    )(qd, ks, vs, bd)


def _t7_body(h_ref, n0, n1, n2, n3, den_ref, es_ref, v_ref, wo_ref,
             g1_ref, b1_ref, wf1_ref, bf1_ref, wf2_ref, bf2_ref,
             g2_ref, b2_ref, out_ref):
    h = h_ref[...]
    num = jnp.concatenate([r[...] for r in (n0, n1, n2, n3)], axis=1)
    den = den_ref[:, 0:8]
    es = es_ref[...]
    v = v_ref[...]
    dentot = den + es + 1e-30
    segs = []
    for hh in range(HEADS):
        b = hh * DH
        numh = num[:, b:b + DH] + es[:, hh:hh + 1] * v[:, b:b + DH]
        segs.append(numh / dentot[:, hh:hh + 1])
    attn = jnp.concatenate(segs, axis=1)
    o = h + _dot(attn, wo_ref[...])
    u = _ln(o, g1_ref[...], b1_ref[...])
    ff = _dot(jnp.maximum(_dot(u, wf1_ref[...]) + bf1_ref[...], 0.0),
              wf2_ref[...]) + bf2_ref[...]
    out_ref[...] = _ln(u + ff, g2_ref[...], b2_ref[...])


def _t7(h, num_chunks, den, es, v, Wo, g1, b1, Wf1, bf1, Wf2, bf2, g2, b2):
    return pl.pallas_call(
        _t7_body,
        grid=(N // BLK7,),
        in_specs=[pl.BlockSpec((BLK7, D), lambda i: (i, 0))]
        + [pl.BlockSpec((BLK7, 128), lambda i: (i, 0))] * 4
        + [
            pl.BlockSpec((BLK7, 16), lambda i: (i, 0)),
            pl.BlockSpec((BLK7, 8), lambda i: (i, 0)),
            pl.BlockSpec((BLK7, D), lambda i: (i, 0)),
            pl.BlockSpec((D, D), lambda i: (0, 0)),
            pl.BlockSpec((1, D), lambda i: (0, 0)),
            pl.BlockSpec((1, D), lambda i: (0, 0)),
            pl.BlockSpec((D, 4 * D), lambda i: (0, 0)),
            pl.BlockSpec((1, 4 * D), lambda i: (0, 0)),
            pl.BlockSpec((4 * D, D), lambda i: (0, 0)),
            pl.BlockSpec((1, D), lambda i: (0, 0)),
            pl.BlockSpec((1, D), lambda i: (0, 0)),
            pl.BlockSpec((1, D), lambda i: (0, 0)),
        ],
        out_specs=pl.BlockSpec((BLK7, D), lambda i: (i, 0)),
        out_shape=jax.ShapeDtypeStruct((N, D), jnp.float32),
    )(h, *num_chunks, den, es, v, Wo, g1, b1, Wf1, bf1, Wf2, bf2, g2, b2)


def _t8_body(h_ref, w1_ref, b1_ref, w2_ref, b2_ref, out_ref):
    t = jnp.maximum(_dot(h_ref[...], w1_ref[...]) + b1_ref[...], 0.0)
    out_ref[...] = _dot(t, w2_ref[...]) + b2_ref[...]


def _t8(h, Wp1, bp1, Wp2, bp2):
    return pl.pallas_call(
        _t8_body,
        grid=(N // BLK,),
        in_specs=[
            pl.BlockSpec((BLK, D), lambda i: (i, 0)),
            pl.BlockSpec((D, D // 2), lambda i: (0, 0)),
            pl.BlockSpec((1, D // 2), lambda i: (0, 0)),
            pl.BlockSpec((D // 2, D), lambda i: (0, 0)),
            pl.BlockSpec((1, D), lambda i: (0, 0)),
        ],
        out_specs=pl.BlockSpec((BLK, D), lambda i: (i, 0)),
        out_shape=jax.ShapeDtypeStruct((N, D), jnp.float32),
    )(h, Wp1, bp1, Wp2, bp2)


def kernel(x, W_gcn1, b_gcn1, W_gcn2, b_gcn2, Wq, Wk, Wv, Wo, ln1_g, ln1_b,
           W_ff1, b_ff1, W_ff2, b_ff2, ln2_g, ln2_b, Wp1, bp1, Wp2, bp2,
           edge_index):
    src = edge_index[0]
    dst = edge_index[1]
    perm = jnp.argsort(dst)
    src_s = src[perm]
    dst_s = dst[perm]
    starts = jnp.searchsorted(dst_s, jnp.arange(0, ACC + 1, RPT,
                                                dtype=dst.dtype)).astype(jnp.int32)
    aw = (starts[:NW] // 128) * 128
    bw = ((starts[1:NW + 1] + 127) // 128) * 128
    nb = jnp.minimum((bw - aw) // 128, MAXNB)
    params = jnp.pad(jnp.stack([aw, nb], axis=1), ((0, 0), (0, 14)))
    src_sp = jnp.pad(src_s, (0, EPAD))
    dst_sp = jnp.pad(dst_s, (0, EPAD), constant_values=ACC)

    bd = jnp.asarray(
        (np.arange(D)[:, None] // DH == np.arange(HEADS)[None, :])
        .astype(np.float32))

    def r2(a):
        return a.reshape(1, -1)

    degp = _sc_deg(dst_sp, params)[:N]

    *hwp1, dinv = _t1(x, W_gcn1, degp)
    agg1 = [_sc_gather_sum(hwp1[j], src_sp, dst_sp, params)[:N]
            for j in range(4)]
    h = _t2(agg1, hwp1, dinv, r2(b_gcn1))

    hwp2 = _t3(h, W_gcn2, dinv)
    agg2 = [_sc_gather_sum(hwp2[j], src_sp, dst_sp, params)[:N]
            for j in range(4)]
    h = _t2(agg2, hwp2, dinv, r2(b_gcn2))

    for l in range(LAYERS):
        q, k, v, es, qb, kb, vb = _t5(h, Wq[l], Wk[l], Wv[l], bd)

        def pack32(a):
            return jax.lax.bitcast_convert_type(
                a.reshape(N, D // 2, 2), jnp.int32)

        def unpack16(a):
            return jax.lax.bitcast_convert_type(a, jnp.bfloat16).reshape(E, D)

        qd, ks, vs = _sc_gather3(pack32(qb), pack32(kb), pack32(vb),
                                 src_s, dst_s)
        qd, ks, vs = unpack16(qd), unpack16(ks), unpack16(vs)
        ex16, w0, w1, w2, w3 = _t6(qd, ks, vs, bd)
        den = _sc_scatter_sum(ex16, dst_sp, params)[:N]
        nump = [_sc_scatter_sum(wj, dst_sp, params)[:N]
                for wj in (w0, w1, w2, w3)]
        h = _t7(h, nump, den, es, v, Wo[l], r2(ln1_g[l]), r2(ln1_b[l]),
                W_ff1[l], r2(b_ff1[l]), W_ff2[l], r2(b_ff2[l]),
                r2(ln2_g[l]), r2(ln2_b[l]))

    x_pred = _t8(h, Wp1, r2(bp1), Wp2, r2(bp2))
    return (h, x_pred)


# speedup vs baseline: 2.4769x; 2.4769x over previous
import functools

import jax
import jax.numpy as jnp
import numpy as np
from jax import lax
from jax.experimental import pallas as pl
from jax.experimental.pallas import tpu as pltpu
from jax.experimental.pallas import tpu_sc as plsc

N = 10000
E = 160000
D_IN = 256
D = 512
HEADS = 8
DH = 64
LAYERS = 2

NC = 2
NS = 16
NW = NC * NS
ACC = 10240
RPT = ACC // NW
MAXNB = 96
EPAD = MAXNB * 128

_MESH = dict(core_axis_name="c", subcore_axis_name="s", num_cores=NC,
             num_subcores=NS)


def _fill_vmem(ref, rows, cols, value):
    nseg = cols // 16

    def body(i, _):
        r = i // nseg
        cc = (i % nseg) * 16
        ref[r, pl.ds(cc, 16)] = jnp.full((16,), value, jnp.float32)
        return 0

    lax.fori_loop(0, rows * nseg, body, 0)


def _make_sorted_sum(C, mode):
    scratch = [
        pltpu.VMEM((16,), jnp.int32),
        pltpu.VMEM((EPAD,), jnp.int32),
        pltpu.VMEM((RPT + 16, C), jnp.float32),
    ]
    if mode == "gather":
        scratch.append(pltpu.VMEM((EPAD,), jnp.int32))
    if mode != "ones":
        scratch += [pltpu.VMEM((128, C), jnp.float32),
                    pltpu.VMEM((128, C), jnp.float32),
                    pltpu.SemaphoreType.DMA,
                    pltpu.SemaphoreType.DMA]

    def body(*refs):
        if mode == "gather":
            (tbl_hbm, srcs_hbm, dsts_hbm, params_hbm, out_hbm,
             pbuf, idxd, acc, idxs, buf0, buf1, sem0, sem1) = refs
        elif mode == "linear":
            (vals_hbm, dsts_hbm, params_hbm, out_hbm,
             pbuf, idxd, acc, buf0, buf1, sem0, sem1) = refs
        else:
            (dsts_hbm, params_hbm, out_hbm, pbuf, idxd, acc) = refs
        c = lax.axis_index("c")
        s = lax.axis_index("s")
        w = c * NS + s
        base = w * RPT
        pltpu.sync_copy(params_hbm.at[w], pbuf)
        pv = pbuf[...]
        aw = pl.multiple_of(pv[0], 128)
        nb = pv[1]
        _fill_vmem(acc, RPT + 16, C, 0.0)
        pltpu.sync_copy(dsts_hbm.at[pl.ds(aw, EPAD)], idxd)
        if mode == "gather":
            pltpu.sync_copy(srcs_hbm.at[pl.ds(aw, EPAD)], idxs)

        ones = jnp.full((16,), 1.0, jnp.float32)

        def process(b, rbuf):
            for g in range(8):
                dstv = idxd[pl.ds(pl.multiple_of(b * 128 + g * 16, 16), 16)]
                rel = dstv - base
                ok = (rel >= 0) & (rel < RPT)
                rr = jnp.where(ok, rel, RPT)
                if mode == "ones":
                    for j in range(16):
                        plsc.addupdate(acc.at[rr[j], pl.ds(0, 16)], ones)
                else:
                    for j in range(0, 16, 2):
                        r0, r1 = rr[j], rr[j + 1]
                        vals0 = [rbuf[g * 16 + j, pl.ds(seg * 16, 16)]
                                 for seg in range(C // 16)]
                        vals1 = [rbuf[g * 16 + j + 1, pl.ds(seg * 16, 16)]
                                 for seg in range(C // 16)]
                        for seg in range(C // 16):
                            plsc.addupdate(acc.at[r0, pl.ds(seg * 16, 16)],
                                           vals0[seg])
                        for seg in range(C // 16):
                            plsc.addupdate(acc.at[r1, pl.ds(seg * 16, 16)],
                                           vals1[seg])

        if mode == "ones":
            def bbody(b, _):
                process(b, None)
                return 0
            lax.fori_loop(0, nb, bbody, 0)
        else:
            bufs = (buf0, buf1)
            sems = (sem0, sem1)

            def mk(b, p):
                if mode == "gather":
                    return pltpu.make_async_copy(
                        tbl_hbm.at[idxs.at[pl.ds(pl.multiple_of(b * 128, 128),
                                                 128)]],
                        bufs[p], sems[p])
                return pltpu.make_async_copy(
                    vals_hbm.at[pl.ds(pl.multiple_of(aw + b * 128, 128),
                                      128), :],
                    bufs[p], sems[p])

            @pl.when(nb > 0)
            def _():
                mk(0, 0).start()

            @pl.when(nb > 1)
            def _():
                mk(1, 1).start()

            def bbody(i, _):
                b0 = 2 * i
                b1 = b0 + 1

                @pl.when(b0 < nb)
                def _():
                    mk(b0, 0).wait()
                    process(b0, bufs[0])

                    @pl.when(b0 + 2 < nb)
                    def _():
                        mk(b0 + 2, 0).start()

                @pl.when(b1 < nb)
                def _():
                    mk(b1, 1).wait()
                    process(b1, bufs[1])

                    @pl.when(b1 + 2 < nb)
                    def _():
                        mk(b1 + 2, 1).start()

                return 0

            lax.fori_loop(0, (MAXNB + 1) // 2, bbody, 0)

        pltpu.sync_copy(acc.at[pl.ds(0, RPT), :],
                        out_hbm.at[pl.ds(w * RPT, RPT), :])

    def outs(_):
        return jax.ShapeDtypeStruct((ACC, C), jnp.float32)

    return pl.kernel(
        body,
        out_type=jax.ShapeDtypeStruct((ACC, C), jnp.float32),
        mesh=plsc.VectorSubcoreMesh(**_MESH),
        scratch_types=scratch,
    )


_make_sorted_sum = functools.lru_cache(maxsize=None)(_make_sorted_sum)


def _sc_deg(dst_s, params):
    return _make_sorted_sum(16, "ones")(dst_s, params)


def _sc_gather_sum(tbl, src_s, dst_s, params):
    return _make_sorted_sum(128, "gather")(tbl, src_s, dst_s, params)


def _sc_scatter_sum(vals, dst_s, params):
    return _make_sorted_sum(vals.shape[1], "linear")(vals, dst_s, params)


_GB = 40
_GNB = 125
_EPW = 5000


@functools.lru_cache(maxsize=None)
def _gather3_kernel():
    def body(q_hbm, k_hbm, v_hbm, srcs_hbm, dsts_hbm,
             qd_hbm, ks_hbm, vs_hbm,
             idxs, idxd, b0, b1, b2, b3,
             g0, g1, g2, g3, w0, w1, w2, w3):
        c = lax.axis_index("c")
        s = lax.axis_index("s")
        w = c * NS + s
        e0 = pl.multiple_of(w * _EPW, 8)
        pltpu.sync_copy(srcs_hbm.at[pl.ds(e0, _EPW)], idxs)
        pltpu.sync_copy(dsts_hbm.at[pl.ds(e0, _EPW)], idxd)
        bufs = (b0, b1, b2, b3)
        gsems = (g0, g1, g2, g3)
        wsems = (w0, w1, w2, w3)

        def one_pass(tbl, idx, out):
            def mkg(b, p):
                return pltpu.make_async_copy(
                    tbl.at[idx.at[pl.ds(pl.multiple_of(b * _GB, 8), _GB)]],
                    bufs[p], gsems[p])

            def mkw(b, p):
                return pltpu.make_async_copy(
                    bufs[p],
                    out.at[pl.ds(pl.multiple_of(e0 + b * _GB, 8), _GB), :],
                    wsems[p])

            mkg(0, 0).start()
            mkg(1, 1).start()

            def bbody(i, _):
                for k4 in range(4):
                    b = 4 * i + k4
                    p = k4

                    @pl.when(b < _GNB)
                    def _():
                        mkg(b, p).wait()
                        mkw(b, p).start()
                        nxt = b + 2
                        pn = (k4 + 2) % 4

                        @pl.when(nxt < _GNB)
                        def _():

                            @pl.when(nxt >= 4)
                            def _():
                                mkw(nxt - 4, pn).wait()

                            mkg(nxt, pn).start()

                return 0

            lax.fori_loop(0, (_GNB + 3) // 4, bbody, 0)
            for tail in range(4):
                b = _GNB - 4 + tail
                mkw(b, b % 4).wait()

        one_pass(q_hbm, idxd, qd_hbm)
        one_pass(k_hbm, idxs, ks_hbm)
        one_pass(v_hbm, idxs, vs_hbm)

    return pl.kernel(
        body,
        out_type=(jax.ShapeDtypeStruct((E, D // 2), jnp.int32),
                  jax.ShapeDtypeStruct((E, D // 2), jnp.int32),
                  jax.ShapeDtypeStruct((E, D // 2), jnp.int32)),
        mesh=plsc.VectorSubcoreMesh(**_MESH),
        scratch_types=[
            pltpu.VMEM((_EPW,), jnp.int32),
            pltpu.VMEM((_EPW,), jnp.int32),
        ] + [pltpu.VMEM((_GB, D // 2), jnp.int32)] * 4
        + [pltpu.SemaphoreType.DMA] * 8,
    )


def _sc_gather3(q, k, v, src_s, dst_s):
    return _gather3_kernel()(q, k, v, src_s, dst_s)


BLK = 2000
BLK7 = 1000
EBLK = 2000


def _ln(x, g, b):
    mu = jnp.mean(x, axis=-1, keepdims=True)
    xc = x - mu
    var = jnp.mean(xc * xc, axis=-1, keepdims=True)
    return xc * lax.rsqrt(var + 1e-5) * g + b


def _dot(a, b):
    return jnp.dot(a, b, preferred_element_type=jnp.float32)


def _t1_body(x_ref, w_ref, degp_ref, c0, c1, c2, c3, dinv_ref):
    xw = _dot(x_ref[...], w_ref[...])
    deg = degp_ref[:, 0:1] + 1.0
    dinv = lax.rsqrt(deg)
    hwp = xw * dinv
    outs = (c0, c1, c2, c3)
    for j in range(4):
        outs[j][...] = hwp[:, j * 128:(j + 1) * 128]
    dinv_ref[...] = dinv


def _t1(x, W1, degp):
    return pl.pallas_call(
        _t1_body,
        grid=(N // BLK,),
        in_specs=[
            pl.BlockSpec((BLK, D_IN), lambda i: (i, 0)),
            pl.BlockSpec((D_IN, D), lambda i: (0, 0)),
            pl.BlockSpec((BLK, 16), lambda i: (i, 0)),
        ],
        out_specs=[pl.BlockSpec((BLK, 128), lambda i: (i, 0))] * 4
        + [pl.BlockSpec((BLK, 1), lambda i: (i, 0))],
        out_shape=[jax.ShapeDtypeStruct((N, 128), jnp.float32)] * 4
        + [jax.ShapeDtypeStruct((N, 1), jnp.float32)],
    )(x, W1, degp)


def _t3_body(h_ref, w_ref, dinv_ref, c0, c1, c2, c3):
    hw = _dot(h_ref[...], w_ref[...])
    hwp = hw * dinv_ref[...]
    outs = (c0, c1, c2, c3)
    for j in range(4):
        outs[j][...] = hwp[:, j * 128:(j + 1) * 128]


def _t3(h, W2, dinv):
    return pl.pallas_call(
        _t3_body,
        grid=(N // BLK,),
        in_specs=[
            pl.BlockSpec((BLK, D), lambda i: (i, 0)),
            pl.BlockSpec((D, D), lambda i: (0, 0)),
            pl.BlockSpec((BLK, 1), lambda i: (i, 0)),
        ],
        out_specs=[pl.BlockSpec((BLK, 128), lambda i: (i, 0))] * 4,
        out_shape=[jax.ShapeDtypeStruct((N, 128), jnp.float32)] * 4,
    )(h, W2, dinv)


def _t2_body(a0, a1, a2, a3, c0, c1, c2, c3, dinv_ref, b_ref, out_ref):
    agg = jnp.concatenate([r[...] for r in (a0, a1, a2, a3)], axis=1)
    hwp = jnp.concatenate([r[...] for r in (c0, c1, c2, c3)], axis=1)
    out_ref[...] = jax.nn.gelu(dinv_ref[...] * (agg + hwp) + b_ref[...])


def _t2(agg_chunks, hwp_chunks, dinv, bias):
    return pl.pallas_call(
        _t2_body,
        grid=(N // BLK,),
        in_specs=[pl.BlockSpec((BLK, 128), lambda i: (i, 0))] * 8
        + [
            pl.BlockSpec((BLK, 1), lambda i: (i, 0)),
            pl.BlockSpec((1, D), lambda i: (0, 0)),
        ],
        out_specs=pl.BlockSpec((BLK, D), lambda i: (i, 0)),
        out_shape=jax.ShapeDtypeStruct((N, D), jnp.float32),
    )(*agg_chunks, *hwp_chunks, dinv, bias)


_MASKHI = np.int32(-65536)


def _pack_bf16(even, odd):
    be = lax.bitcast_convert_type(even, jnp.int32)
    bo = lax.bitcast_convert_type(odd, jnp.int32)
    lo = ((be + 0x8000) >> 16) & 0xFFFF
    hi = (bo + 0x8000) & _MASKHI
    return hi | lo


def _unpack_bf16(p32):
    even = lax.bitcast_convert_type(p32 << 16, jnp.float32)
    odd = lax.bitcast_convert_type(p32 & _MASKHI, jnp.float32)
    return even, odd


def _t5_body(h_ref, wqe_ref, wqo_ref, wke_ref, wko_ref, wve_ref, wvo_ref,
             wvp_ref, bdh_ref, es_ref, vp_ref, qb_ref, kb_ref, vb_ref):
    h = h_ref[...]
    qe = _dot(h, wqe_ref[...])
    qo = _dot(h, wqo_ref[...])
    ke = _dot(h, wke_ref[...])
    ko = _dot(h, wko_ref[...])
    ve = _dot(h, wve_ref[...])
    vo = _dot(h, wvo_ref[...])
    vp_ref[...] = _dot(h, wvp_ref[...])
    es_ref[...] = jnp.exp(_dot(qe * ke + qo * ko, bdh_ref[...]) * 0.125)
    qb_ref[...] = _pack_bf16(qe, qo)
    kb_ref[...] = _pack_bf16(ke, ko)
    vb_ref[...] = _pack_bf16(ve, vo)


def _t5(h, Wqe, Wqo, Wke, Wko, Wve, Wvo, Wvp, bdh):
    return pl.pallas_call(
        _t5_body,
        grid=(N // BLK,),
        in_specs=[pl.BlockSpec((BLK, D), lambda i: (i, 0))]
        + [pl.BlockSpec((D, D // 2), lambda i: (0, 0))] * 6
        + [
            pl.BlockSpec((D, D), lambda i: (0, 0)),
            pl.BlockSpec((D // 2, 8), lambda i: (0, 0)),
        ],
        out_specs=[
            pl.BlockSpec((BLK, 8), lambda i: (i, 0)),
            pl.BlockSpec((BLK, D), lambda i: (i, 0)),
        ] + [pl.BlockSpec((BLK, D // 2), lambda i: (i, 0))] * 3,
        out_shape=[
            jax.ShapeDtypeStruct((N, 8), jnp.float32),
            jax.ShapeDtypeStruct((N, D), jnp.float32),
        ] + [jax.ShapeDtypeStruct((N, D // 2), jnp.int32)] * 3,
    )(h, Wqe, Wqo, Wke, Wko, Wve, Wvo, Wvp, bdh)


def _t6_body(qd_ref, ks_ref, vs_ref, bdh_ref, ex_ref, w0, w1, w2, w3):
    qe, qo = _unpack_bf16(qd_ref[...])
    ke, ko = _unpack_bf16(ks_ref[...])
    ex = jnp.exp(_dot(qe * ke + qo * ko, bdh_ref[...]) * 0.125)
    ex_ref[...] = jnp.concatenate([ex, ex], axis=1)
    ve, vo = _unpack_bf16(vs_ref[...])
    outs = (w0, w1, w2, w3)
    for j in range(4):
        cA = 32 * (2 * j)
        cB = 32 * (2 * j + 1)
        exA = ex[:, 2 * j:2 * j + 1]
        exB = ex[:, 2 * j + 1:2 * j + 2]
        outs[j][...] = jnp.concatenate(
            [ve[:, cA:cA + 32] * exA, vo[:, cA:cA + 32] * exA,
             ve[:, cB:cB + 32] * exB, vo[:, cB:cB + 32] * exB], axis=1)


def _t6(qd, ks, vs, bdh):
    return pl.pallas_call(
        _t6_body,
        grid=(E // EBLK,),
        in_specs=[
            pl.BlockSpec((EBLK, D // 2), lambda i: (i, 0)),
            pl.BlockSpec((EBLK, D // 2), lambda i: (i, 0)),
            pl.BlockSpec((EBLK, D // 2), lambda i: (i, 0)),
            pl.BlockSpec((D // 2, 8), lambda i: (0, 0)),
        ],
        out_specs=[pl.BlockSpec((EBLK, 16), lambda i: (i, 0))]
        + [pl.BlockSpec((EBLK, 128), lambda i: (i, 0))] * 4,
        out_shape=[jax.ShapeDtypeStruct((E, 16), jnp.float32)]
        + [jax.ShapeDtypeStruct((E, 128), jnp.float32)] * 4,
    )(qd, ks, vs, bdh)


def _t7_body(h_ref, n0, n1, n2, n3, den_ref, es_ref, v_ref, wo_ref,
             g1_ref, b1_ref, wf1_ref, bf1_ref, wf2_ref, bf2_ref,
             g2_ref, b2_ref, out_ref):
    h = h_ref[...]
    num = jnp.concatenate([r[...] for r in (n0, n1, n2, n3)], axis=1)
    den = den_ref[:, 0:8]
    es = es_ref[...]
    v = v_ref[...]
    dentot = den + es + 1e-30
    segs = []
    for hh in range(HEADS):
        b = hh * DH
        numh = num[:, b:b + DH] + es[:, hh:hh + 1] * v[:, b:b + DH]
        segs.append(numh / dentot[:, hh:hh + 1])
    attn = jnp.concatenate(segs, axis=1)
    o = h + _dot(attn, wo_ref[...])
    u = _ln(o, g1_ref[...], b1_ref[...])
    ff = _dot(jnp.maximum(_dot(u, wf1_ref[...]) + bf1_ref[...], 0.0),
              wf2_ref[...]) + bf2_ref[...]
    out_ref[...] = _ln(u + ff, g2_ref[...], b2_ref[...])


def _t7(h, num_chunks, den, es, v, Wo, g1, b1, Wf1, bf1, Wf2, bf2, g2, b2):
    return pl.pallas_call(
        _t7_body,
        grid=(N // BLK7,),
        in_specs=[pl.BlockSpec((BLK7, D), lambda i: (i, 0))]
        + [pl.BlockSpec((BLK7, 128), lambda i: (i, 0))] * 4
        + [
            pl.BlockSpec((BLK7, 16), lambda i: (i, 0)),
            pl.BlockSpec((BLK7, 8), lambda i: (i, 0)),
            pl.BlockSpec((BLK7, D), lambda i: (i, 0)),
            pl.BlockSpec((D, D), lambda i: (0, 0)),
            pl.BlockSpec((1, D), lambda i: (0, 0)),
            pl.BlockSpec((1, D), lambda i: (0, 0)),
            pl.BlockSpec((D, 4 * D), lambda i: (0, 0)),
            pl.BlockSpec((1, 4 * D), lambda i: (0, 0)),
            pl.BlockSpec((4 * D, D), lambda i: (0, 0)),
            pl.BlockSpec((1, D), lambda i: (0, 0)),
            pl.BlockSpec((1, D), lambda i: (0, 0)),
            pl.BlockSpec((1, D), lambda i: (0, 0)),
        ],
        out_specs=pl.BlockSpec((BLK7, D), lambda i: (i, 0)),
        out_shape=jax.ShapeDtypeStruct((N, D), jnp.float32),
    )(h, *num_chunks, den, es, v, Wo, g1, b1, Wf1, bf1, Wf2, bf2, g2, b2)


def _t8_body(h_ref, w1_ref, b1_ref, w2_ref, b2_ref, out_ref):
    t = jnp.maximum(_dot(h_ref[...], w1_ref[...]) + b1_ref[...], 0.0)
    out_ref[...] = _dot(t, w2_ref[...]) + b2_ref[...]


def _t8(h, Wp1, bp1, Wp2, bp2):
    return pl.pallas_call(
        _t8_body,
        grid=(N // BLK,),
        in_specs=[
            pl.BlockSpec((BLK, D), lambda i: (i, 0)),
            pl.BlockSpec((D, D // 2), lambda i: (0, 0)),
            pl.BlockSpec((1, D // 2), lambda i: (0, 0)),
            pl.BlockSpec((D // 2, D), lambda i: (0, 0)),
            pl.BlockSpec((1, D), lambda i: (0, 0)),
        ],
        out_specs=pl.BlockSpec((BLK, D), lambda i: (i, 0)),
        out_shape=jax.ShapeDtypeStruct((N, D), jnp.float32),
    )(h, Wp1, bp1, Wp2, bp2)


def kernel(x, W_gcn1, b_gcn1, W_gcn2, b_gcn2, Wq, Wk, Wv, Wo, ln1_g, ln1_b,
           W_ff1, b_ff1, W_ff2, b_ff2, ln2_g, ln2_b, Wp1, bp1, Wp2, bp2,
           edge_index):
    src = edge_index[0]
    dst = edge_index[1]
    perm = jnp.argsort(dst)
    src_s = src[perm]
    dst_s = dst[perm]
    starts = jnp.searchsorted(dst_s, jnp.arange(0, ACC + 1, RPT,
                                                dtype=dst.dtype)).astype(jnp.int32)
    aw = (starts[:NW] // 128) * 128
    bw = ((starts[1:NW + 1] + 127) // 128) * 128
    nb = jnp.minimum((bw - aw) // 128, MAXNB)
    params = jnp.pad(jnp.stack([aw, nb], axis=1), ((0, 0), (0, 14)))
    src_sp = jnp.pad(src_s, (0, EPAD))
    dst_sp = jnp.pad(dst_s, (0, EPAD), constant_values=ACC)

    bdh = jnp.asarray(
        (np.arange(D // 2)[:, None] // (DH // 2) == np.arange(HEADS)[None, :])
        .astype(np.float32))
    pcols = np.concatenate([
        np.concatenate([64 * h + 2 * np.arange(32),
                        64 * h + 2 * np.arange(32) + 1])
        for h in range(HEADS)])

    def r2(a):
        return a.reshape(1, -1)

    degp = _sc_deg(dst_sp, params)[:N]

    *hwp1, dinv = _t1(x, W_gcn1, degp)
    agg1 = [_sc_gather_sum(hwp1[j], src_sp, dst_sp, params)[:N]
            for j in range(4)]
    h = _t2(agg1, hwp1, dinv, r2(b_gcn1))

    hwp2 = _t3(h, W_gcn2, dinv)
    agg2 = [_sc_gather_sum(hwp2[j], src_sp, dst_sp, params)[:N]
            for j in range(4)]
    h = _t2(agg2, hwp2, dinv, r2(b_gcn2))

    for l in range(LAYERS):
        es, vp, qb, kb, vb = _t5(
            h, Wq[l][:, 0::2], Wq[l][:, 1::2], Wk[l][:, 0::2], Wk[l][:, 1::2],
            Wv[l][:, 0::2], Wv[l][:, 1::2], Wv[l][:, pcols], bdh)
        qd, ks, vs = _sc_gather3(qb, kb, vb, src_s, dst_s)
        ex16, w0, w1, w2, w3 = _t6(qd, ks, vs, bdh)
        den = _sc_scatter_sum(ex16, dst_sp, params)[:N]
        nump = [_sc_scatter_sum(wj, dst_sp, params)[:N]
                for wj in (w0, w1, w2, w3)]
        h = _t7(h, nump, den, es, vp, Wo[l][pcols, :], r2(ln1_g[l]),
                r2(ln1_b[l]), W_ff1[l], r2(b_ff1[l]), W_ff2[l], r2(b_ff2[l]),
                r2(ln2_g[l]), r2(ln2_b[l]))

    x_pred = _t8(h, Wp1, r2(bp1), Wp2, r2(bp2))
    return (h, x_pred)


# speedup vs baseline: 2.4813x; 1.0018x over previous
import functools

import jax
import jax.numpy as jnp
import numpy as np
from jax import lax
from jax.experimental import pallas as pl
from jax.experimental.pallas import tpu as pltpu
from jax.experimental.pallas import tpu_sc as plsc

N = 10000
E = 160000
D_IN = 256
D = 512
HEADS = 8
DH = 64
LAYERS = 2

NC = 2
NS = 16
NW = NC * NS
ACC = 10240
RPT = ACC // NW
MAXNB = 96
EPAD = MAXNB * 128

_MESH = dict(core_axis_name="c", subcore_axis_name="s", num_cores=NC,
             num_subcores=NS)


def _fill_vmem(ref, rows, cols, value):
    nseg = cols // 16

    def body(i, _):
        r = i // nseg
        cc = (i % nseg) * 16
        ref[r, pl.ds(cc, 16)] = jnp.full((16,), value, jnp.float32)
        return 0

    lax.fori_loop(0, rows * nseg, body, 0)


def _make_sorted_sum(C, mode):
    scratch = [
        pltpu.VMEM((16,), jnp.int32),
        pltpu.VMEM((EPAD,), jnp.int32),
        pltpu.VMEM((RPT + 16, C), jnp.float32),
    ]
    if mode == "gather":
        scratch.append(pltpu.VMEM((EPAD,), jnp.int32))
    if mode != "ones":
        scratch += [pltpu.VMEM((128, C), jnp.float32),
                    pltpu.VMEM((128, C), jnp.float32),
                    pltpu.SemaphoreType.DMA,
                    pltpu.SemaphoreType.DMA]

    def body(*refs):
        if mode == "gather":
            (tbl_hbm, srcs_hbm, dsts_hbm, params_hbm, out_hbm,
             pbuf, idxd, acc, idxs, buf0, buf1, sem0, sem1) = refs
        elif mode == "linear":
            (vals_hbm, dsts_hbm, params_hbm, out_hbm,
             pbuf, idxd, acc, buf0, buf1, sem0, sem1) = refs
        else:
            (dsts_hbm, params_hbm, out_hbm, pbuf, idxd, acc) = refs
        c = lax.axis_index("c")
        s = lax.axis_index("s")
        w = c * NS + s
        base = w * RPT
        pltpu.sync_copy(params_hbm.at[w], pbuf)
        pv = pbuf[...]
        aw = pl.multiple_of(pv[0], 128)
        nb = pv[1]
        _fill_vmem(acc, RPT + 16, C, 0.0)
        pltpu.sync_copy(dsts_hbm.at[pl.ds(aw, EPAD)], idxd)
        if mode == "gather":
            pltpu.sync_copy(srcs_hbm.at[pl.ds(aw, EPAD)], idxs)

        ones = jnp.full((16,), 1.0, jnp.float32)

        def process(b, rbuf):
            for g in range(8):
                dstv = idxd[pl.ds(pl.multiple_of(b * 128 + g * 16, 16), 16)]
                rel = dstv - base
                ok = (rel >= 0) & (rel < RPT)
                rr = jnp.where(ok, rel, RPT)
                if mode == "ones":
                    for j in range(16):
                        plsc.addupdate(acc.at[rr[j], pl.ds(0, 16)], ones)
                else:
                    for j in range(0, 16, 2):
                        r0, r1 = rr[j], rr[j + 1]
                        vals0 = [rbuf[g * 16 + j, pl.ds(seg * 16, 16)]
                                 for seg in range(C // 16)]
                        vals1 = [rbuf[g * 16 + j + 1, pl.ds(seg * 16, 16)]
                                 for seg in range(C // 16)]
                        for seg in range(C // 16):
                            plsc.addupdate(acc.at[r0, pl.ds(seg * 16, 16)],
                                           vals0[seg])
                        for seg in range(C // 16):
                            plsc.addupdate(acc.at[r1, pl.ds(seg * 16, 16)],
                                           vals1[seg])

        if mode == "ones":
            def bbody(b, _):
                process(b, None)
                return 0
            lax.fori_loop(0, nb, bbody, 0)
        else:
            bufs = (buf0, buf1)
            sems = (sem0, sem1)

            def mk(b, p):
                if mode == "gather":
                    return pltpu.make_async_copy(
                        tbl_hbm.at[idxs.at[pl.ds(pl.multiple_of(b * 128, 128),
                                                 128)]],
                        bufs[p], sems[p])
                return pltpu.make_async_copy(
                    vals_hbm.at[pl.ds(pl.multiple_of(aw + b * 128, 128),
                                      128), :],
                    bufs[p], sems[p])

            @pl.when(nb > 0)
            def _():
                mk(0, 0).start()

            @pl.when(nb > 1)
            def _():
                mk(1, 1).start()

            def bbody(i, _):
                b0 = 2 * i
                b1 = b0 + 1

                @pl.when(b0 < nb)
                def _():
                    mk(b0, 0).wait()
                    process(b0, bufs[0])

                    @pl.when(b0 + 2 < nb)
                    def _():
                        mk(b0 + 2, 0).start()

                @pl.when(b1 < nb)
                def _():
                    mk(b1, 1).wait()
                    process(b1, bufs[1])

                    @pl.when(b1 + 2 < nb)
                    def _():
                        mk(b1 + 2, 1).start()

                return 0

            lax.fori_loop(0, (MAXNB + 1) // 2, bbody, 0)

        pltpu.sync_copy(acc.at[pl.ds(0, RPT), :],
                        out_hbm.at[pl.ds(w * RPT, RPT), :])

    return pl.kernel(
        body,
        out_type=jax.ShapeDtypeStruct((ACC, C), jnp.float32),
        mesh=plsc.VectorSubcoreMesh(**_MESH),
        scratch_types=scratch,
    )


_make_sorted_sum = functools.lru_cache(maxsize=None)(_make_sorted_sum)


def _sc_deg(dst_s, params):
    return _make_sorted_sum(16, "ones")(dst_s, params)


def _sc_gather_sum(tbl, src_s, dst_s, params):
    return _make_sorted_sum(128, "gather")(tbl, src_s, dst_s, params)


def _sc_scatter_sum(vals, dst_s, params):
    return _make_sorted_sum(vals.shape[1], "linear")(vals, dst_s, params)


_GB = 40
_GNB = 125
_EPW = 5000


@functools.lru_cache(maxsize=None)
def _gather3_kernel():
    def body(q_hbm, k_hbm, v_hbm, srcs_hbm, dsts_hbm,
             qd_hbm, ks_hbm, vs_hbm,
             idxs, idxd, b0, b1, b2, b3,
             g0, g1, g2, g3, w0, w1, w2, w3):
        c = lax.axis_index("c")
        s = lax.axis_index("s")
        w = c * NS + s
        e0 = pl.multiple_of(w * _EPW, 8)
        pltpu.sync_copy(srcs_hbm.at[pl.ds(e0, _EPW)], idxs)
        pltpu.sync_copy(dsts_hbm.at[pl.ds(e0, _EPW)], idxd)
        bufs = (b0, b1, b2, b3)
        gsems = (g0, g1, g2, g3)
        wsems = (w0, w1, w2, w3)

        def one_pass(tbl, idx, out):
            def mkg(b, p):
                return pltpu.make_async_copy(
                    tbl.at[idx.at[pl.ds(pl.multiple_of(b * _GB, 8), _GB)]],
                    bufs[p], gsems[p])

            def mkw(b, p):
                return pltpu.make_async_copy(
                    bufs[p],
                    out.at[pl.ds(pl.multiple_of(e0 + b * _GB, 8), _GB), :],
                    wsems[p])

            mkg(0, 0).start()
            mkg(1, 1).start()

            def bbody(i, _):
                for k4 in range(4):
                    b = 4 * i + k4
                    p = k4

                    @pl.when(b < _GNB)
                    def _():
                        mkg(b, p).wait()
                        mkw(b, p).start()
                        nxt = b + 2
                        pn = (k4 + 2) % 4

                        @pl.when(nxt < _GNB)
                        def _():

                            @pl.when(nxt >= 4)
                            def _():
                                mkw(nxt - 4, pn).wait()

                            mkg(nxt, pn).start()

                return 0

            lax.fori_loop(0, (_GNB + 3) // 4, bbody, 0)
            for tail in range(4):
                b = _GNB - 4 + tail
                mkw(b, b % 4).wait()

        one_pass(q_hbm, idxd, qd_hbm)
        one_pass(k_hbm, idxs, ks_hbm)
        one_pass(v_hbm, idxs, vs_hbm)

    return pl.kernel(
        body,
        out_type=(jax.ShapeDtypeStruct((E, D // 2), jnp.int32),
                  jax.ShapeDtypeStruct((E, D // 2), jnp.int32),
                  jax.ShapeDtypeStruct((E, D // 2), jnp.int32)),
        mesh=plsc.VectorSubcoreMesh(**_MESH),
        scratch_types=[
            pltpu.VMEM((_EPW,), jnp.int32),
            pltpu.VMEM((_EPW,), jnp.int32),
        ] + [pltpu.VMEM((_GB, D // 2), jnp.int32)] * 4
        + [pltpu.SemaphoreType.DMA] * 8,
    )


def _sc_gather3(q, k, v, src_s, dst_s):
    return _gather3_kernel()(q, k, v, src_s, dst_s)


BLK = 2000
BLK7 = 1000
EBLK = 2000


def _ln(x, g, b):
    mu = jnp.mean(x, axis=-1, keepdims=True)
    xc = x - mu
    var = jnp.mean(xc * xc, axis=-1, keepdims=True)
    return xc * lax.rsqrt(var + 1e-5) * g + b


def _dot(a, b):
    return jnp.dot(a, b, preferred_element_type=jnp.float32)


def _t1_body(x_ref, w_ref, degp_ref, c0, c1, c2, c3, dinv_ref):
    xw = _dot(x_ref[...], w_ref[...])
    deg = degp_ref[:, 0:1] + 1.0
    dinv = lax.rsqrt(deg)
    hwp = xw * dinv
    outs = (c0, c1, c2, c3)
    for j in range(4):
        outs[j][...] = hwp[:, j * 128:(j + 1) * 128]
    dinv_ref[...] = dinv


def _t1(x, W1, degp):
    return pl.pallas_call(
        _t1_body,
        grid=(N // BLK,),
        in_specs=[
            pl.BlockSpec((BLK, D_IN), lambda i: (i, 0)),
            pl.BlockSpec((D_IN, D), lambda i: (0, 0)),
            pl.BlockSpec((BLK, 16), lambda i: (i, 0)),
        ],
        out_specs=[pl.BlockSpec((BLK, 128), lambda i: (i, 0))] * 4
        + [pl.BlockSpec((BLK, 1), lambda i: (i, 0))],
        out_shape=[jax.ShapeDtypeStruct((N, 128), jnp.float32)] * 4
        + [jax.ShapeDtypeStruct((N, 1), jnp.float32)],
    )(x, W1, degp)


def _t3_body(h_ref, w_ref, dinv_ref, c0, c1, c2, c3):
    hw = _dot(h_ref[...], w_ref[...])
    hwp = hw * dinv_ref[...]
    outs = (c0, c1, c2, c3)
    for j in range(4):
        outs[j][...] = hwp[:, j * 128:(j + 1) * 128]


def _t3(h, W2, dinv):
    return pl.pallas_call(
        _t3_body,
        grid=(N // BLK,),
        in_specs=[
            pl.BlockSpec((BLK, D), lambda i: (i, 0)),
            pl.BlockSpec((D, D), lambda i: (0, 0)),
            pl.BlockSpec((BLK, 1), lambda i: (i, 0)),
        ],
        out_specs=[pl.BlockSpec((BLK, 128), lambda i: (i, 0))] * 4,
        out_shape=[jax.ShapeDtypeStruct((N, 128), jnp.float32)] * 4,
    )(h, W2, dinv)


def _t2_body(a0, a1, a2, a3, c0, c1, c2, c3, dinv_ref, b_ref, out_ref):
    agg = jnp.concatenate([r[...] for r in (a0, a1, a2, a3)], axis=1)
    hwp = jnp.concatenate([r[...] for r in (c0, c1, c2, c3)], axis=1)
    out_ref[...] = jax.nn.gelu(dinv_ref[...] * (agg + hwp) + b_ref[...])


def _t2(agg_chunks, hwp_chunks, dinv, bias):
    return pl.pallas_call(
        _t2_body,
        grid=(N // BLK,),
        in_specs=[pl.BlockSpec((BLK, 128), lambda i: (i, 0))] * 8
        + [
            pl.BlockSpec((BLK, 1), lambda i: (i, 0)),
            pl.BlockSpec((1, D), lambda i: (0, 0)),
        ],
        out_specs=pl.BlockSpec((BLK, D), lambda i: (i, 0)),
        out_shape=jax.ShapeDtypeStruct((N, D), jnp.float32),
    )(*agg_chunks, *hwp_chunks, dinv, bias)


_MASKHI = np.int32(-65536)


def _pack_bf16(even, odd):
    be = lax.bitcast_convert_type(even, jnp.int32)
    bo = lax.bitcast_convert_type(odd, jnp.int32)
    lo = ((be + 0x8000) >> 16) & 0xFFFF
    hi = (bo + 0x8000) & _MASKHI
    return hi | lo


def _unpack_bf16(p32):
    even = lax.bitcast_convert_type(p32 << 16, jnp.float32)
    odd = lax.bitcast_convert_type(p32 & _MASKHI, jnp.float32)
    return even, odd


def _t5_body(h_ref, wqe_ref, wqo_ref, wke_ref, wko_ref, wve_ref, wvo_ref,
             wvp_ref, bdh_ref, es_ref, vp_ref, qb_ref, kb_ref, vb_ref):
    h = h_ref[...]
    qe = _dot(h, wqe_ref[...])
    qo = _dot(h, wqo_ref[...])
    ke = _dot(h, wke_ref[...])
    ko = _dot(h, wko_ref[...])
    ve = _dot(h, wve_ref[...])
    vo = _dot(h, wvo_ref[...])
    vp_ref[...] = _dot(h, wvp_ref[...])
    es_ref[...] = jnp.exp(_dot(qe * ke + qo * ko, bdh_ref[...]) * 0.125)
    qb_ref[...] = _pack_bf16(qe, qo)
    kb_ref[...] = _pack_bf16(ke, ko)
    vb_ref[...] = _pack_bf16(ve, vo)


def _t5(h, Wqe, Wqo, Wke, Wko, Wve, Wvo, Wvp, bdh):
    return pl.pallas_call(
        _t5_body,
        grid=(N // BLK,),
        in_specs=[pl.BlockSpec((BLK, D), lambda i: (i, 0))]
        + [pl.BlockSpec((D, D // 2), lambda i: (0, 0))] * 6
        + [
            pl.BlockSpec((D, D), lambda i: (0, 0)),
            pl.BlockSpec((D // 2, 8), lambda i: (0, 0)),
        ],
        out_specs=[
            pl.BlockSpec((BLK, 8), lambda i: (i, 0)),
            pl.BlockSpec((BLK, D), lambda i: (i, 0)),
        ] + [pl.BlockSpec((BLK, D // 2), lambda i: (i, 0))] * 3,
        out_shape=[
            jax.ShapeDtypeStruct((N, 8), jnp.float32),
            jax.ShapeDtypeStruct((N, D), jnp.float32),
        ] + [jax.ShapeDtypeStruct((N, D // 2), jnp.int32)] * 3,
    )(h, Wqe, Wqo, Wke, Wko, Wve, Wvo, Wvp, bdh)


def _t6_body(qd_ref, ks_ref, vs_ref, bdh_ref, ex_ref, w0, w1, w2, w3):
    qe, qo = _unpack_bf16(qd_ref[...])
    ke, ko = _unpack_bf16(ks_ref[...])
    ex = jnp.exp(_dot(qe * ke + qo * ko, bdh_ref[...]) * 0.125)
    ex_ref[...] = jnp.concatenate([ex, ex], axis=1)
    ve, vo = _unpack_bf16(vs_ref[...])
    outs = (w0, w1, w2, w3)
    for j in range(4):
        cA = 32 * (2 * j)
        cB = 32 * (2 * j + 1)
        exA = ex[:, 2 * j:2 * j + 1]
        exB = ex[:, 2 * j + 1:2 * j + 2]
        outs[j][...] = jnp.concatenate(
            [ve[:, cA:cA + 32] * exA, vo[:, cA:cA + 32] * exA,
             ve[:, cB:cB + 32] * exB, vo[:, cB:cB + 32] * exB], axis=1)


def _t6(qd, ks, vs, bdh):
    return pl.pallas_call(
        _t6_body,
        grid=(E // EBLK,),
        in_specs=[
            pl.BlockSpec((EBLK, D // 2), lambda i: (i, 0)),
            pl.BlockSpec((EBLK, D // 2), lambda i: (i, 0)),
            pl.BlockSpec((EBLK, D // 2), lambda i: (i, 0)),
            pl.BlockSpec((D // 2, 8), lambda i: (0, 0)),
        ],
        out_specs=[pl.BlockSpec((EBLK, 16), lambda i: (i, 0))]
        + [pl.BlockSpec((EBLK, 128), lambda i: (i, 0))] * 4,
        out_shape=[jax.ShapeDtypeStruct((E, 16), jnp.float32)]
        + [jax.ShapeDtypeStruct((E, 128), jnp.float32)] * 4,
    )(qd, ks, vs, bdh)


def _t7_body(h_ref, n0, n1, n2, n3, den_ref, es_ref, v_ref, wo_ref,
             g1_ref, b1_ref, wf1_ref, bf1_ref, wf2_ref, bf2_ref,
             g2_ref, b2_ref, out_ref):
    h = h_ref[...]
    num = jnp.concatenate([r[...] for r in (n0, n1, n2, n3)], axis=1)
    den = den_ref[:, 0:8]
    es = es_ref[...]
    v = v_ref[...]
    dentot = den + es + 1e-30
    segs = []
    for hh in range(HEADS):
        b = hh * DH
        numh = num[:, b:b + DH] + es[:, hh:hh + 1] * v[:, b:b + DH]
        segs.append(numh / dentot[:, hh:hh + 1])
    attn = jnp.concatenate(segs, axis=1)
    o = h + _dot(attn, wo_ref[...])
    u = _ln(o, g1_ref[...], b1_ref[...])
    ff = _dot(jnp.maximum(_dot(u, wf1_ref[...]) + bf1_ref[...], 0.0),
              wf2_ref[...]) + bf2_ref[...]
    out_ref[...] = _ln(u + ff, g2_ref[...], b2_ref[...])


def _t7(h, num_chunks, den, es, v, Wo, g1, b1, Wf1, bf1, Wf2, bf2, g2, b2):
    return pl.pallas_call(
        _t7_body,
        grid=(N // BLK7,),
        in_specs=[pl.BlockSpec((BLK7, D), lambda i: (i, 0))]
        + [pl.BlockSpec((BLK7, 128), lambda i: (i, 0))] * 4
        + [
            pl.BlockSpec((BLK7, 16), lambda i: (i, 0)),
            pl.BlockSpec((BLK7, 8), lambda i: (i, 0)),
            pl.BlockSpec((BLK7, D), lambda i: (i, 0)),
            pl.BlockSpec((D, D), lambda i: (0, 0)),
            pl.BlockSpec((1, D), lambda i: (0, 0)),
            pl.BlockSpec((1, D), lambda i: (0, 0)),
            pl.BlockSpec((D, 4 * D), lambda i: (0, 0)),
            pl.BlockSpec((1, 4 * D), lambda i: (0, 0)),
            pl.BlockSpec((4 * D, D), lambda i: (0, 0)),
            pl.BlockSpec((1, D), lambda i: (0, 0)),
            pl.BlockSpec((1, D), lambda i: (0, 0)),
            pl.BlockSpec((1, D), lambda i: (0, 0)),
        ],
        out_specs=pl.BlockSpec((BLK7, D), lambda i: (i, 0)),
        out_shape=jax.ShapeDtypeStruct((N, D), jnp.float32),
    )(h, *num_chunks, den, es, v, Wo, g1, b1, Wf1, bf1, Wf2, bf2, g2, b2)


def _t8_body(h_ref, w1_ref, b1_ref, w2_ref, b2_ref, out_ref):
    t = jnp.maximum(_dot(h_ref[...], w1_ref[...]) + b1_ref[...], 0.0)
    out_ref[...] = _dot(t, w2_ref[...]) + b2_ref[...]


def _t8(h, Wp1, bp1, Wp2, bp2):
    return pl.pallas_call(
        _t8_body,
        grid=(N // BLK,),
        in_specs=[
            pl.BlockSpec((BLK, D), lambda i: (i, 0)),
            pl.BlockSpec((D, D // 2), lambda i: (0, 0)),
            pl.BlockSpec((1, D // 2), lambda i: (0, 0)),
            pl.BlockSpec((D // 2, D), lambda i: (0, 0)),
            pl.BlockSpec((1, D), lambda i: (0, 0)),
        ],
        out_specs=pl.BlockSpec((BLK, D), lambda i: (i, 0)),
        out_shape=jax.ShapeDtypeStruct((N, D), jnp.float32),
    )(h, Wp1, bp1, Wp2, bp2)


def kernel(x, W_gcn1, b_gcn1, W_gcn2, b_gcn2, Wq, Wk, Wv, Wo, ln1_g, ln1_b,
           W_ff1, b_ff1, W_ff2, b_ff2, ln2_g, ln2_b, Wp1, bp1, Wp2, bp2,
           edge_index):
    src = edge_index[0]
    dst = edge_index[1]
    perm = jnp.argsort(dst)
    src_s = src[perm]
    dst_s = dst[perm]
    starts = jnp.searchsorted(dst_s, jnp.arange(0, ACC + 1, RPT,
                                                dtype=dst.dtype)).astype(jnp.int32)
    aw = (starts[:NW] // 128) * 128
    bw = ((starts[1:NW + 1] + 127) // 128) * 128
    nb = jnp.minimum((bw - aw) // 128, MAXNB)
    params = jnp.pad(jnp.stack([aw, nb], axis=1), ((0, 0), (0, 14)))
    src_sp = jnp.pad(src_s, (0, EPAD))
    dst_sp = jnp.pad(dst_s, (0, EPAD), constant_values=ACC)

    bdh = jnp.asarray(
        (np.arange(D // 2)[:, None] // (DH // 2) == np.arange(HEADS)[None, :])
        .astype(np.float32))
    pcols = np.concatenate([
        np.concatenate([64 * h + 2 * np.arange(32),
                        64 * h + 2 * np.arange(32) + 1])
        for h in range(HEADS)])

    def r2(a):
        return a.reshape(1, -1)

    degp = _sc_deg(dst_sp, params)[:N]

    *hwp1, dinv = _t1(x, W_gcn1, degp)
    agg1 = [_sc_gather_sum(hwp1[j], src_sp, dst_sp, params)[:N]
            for j in range(4)]
    h = _t2(agg1, hwp1, dinv, r2(b_gcn1))

    hwp2 = _t3(h, W_gcn2, dinv)
    agg2 = [_sc_gather_sum(hwp2[j], src_sp, dst_sp, params)[:N]
            for j in range(4)]
    h = _t2(agg2, hwp2, dinv, r2(b_gcn2))

    for l in range(LAYERS):
        es, vp, qb, kb, vb = _t5(
            h, Wq[l][:, 0::2], Wq[l][:, 1::2], Wk[l][:, 0::2], Wk[l][:, 1::2],
            Wv[l][:, 0::2], Wv[l][:, 1::2], Wv[l][:, pcols], bdh)
        qd, ks, vs = _sc_gather3(qb, kb, vb, src_s, dst_s)
        ex16, w0, w1, w2, w3 = _t6(qd, ks, vs, bdh)
        den = _sc_scatter_sum(ex16, dst_sp, params)[:N]
        nump = [_sc_scatter_sum(wj, dst_sp, params)[:N]
                for wj in (w0, w1, w2, w3)]
        h = _t7(h, nump, den, es, vp, Wo[l][pcols, :], r2(ln1_g[l]),
                r2(ln1_b[l]), W_ff1[l], r2(b_ff1[l]), W_ff2[l], r2(b_ff2[l]),
                r2(ln2_g[l]), r2(ln2_b[l]))

    x_pred = _t8(h, Wp1, r2(bp1), Wp2, r2(bp2))
    return (h, x_pred)
